# baseline (speedup 1.0000x reference)
import functools

import jax
import jax.numpy as jnp
from jax import lax
from jax.experimental import pallas as pl
from jax.experimental.pallas import tpu as pltpu
from jax.experimental.pallas import tpu_sc as plsc

VOCAB = 32128
D_MODEL = 4096

_NC = 2
_NS = 16
_NW = _NC * _NS

_T = 77
_CHUNK = 8


def _emb_body(idx_hbm, table_hbm, out_hbm, idx_v, rows_v, g0, g1, g2, s0, s1, s2):
    wid = lax.axis_index("s") * _NC + lax.axis_index("c")
    row = wid // 2
    part = wid % 2
    t0 = part * 40
    base = row * _T + t0
    limit = 47 - part * 11

    gsems = (g0, g1, g2)
    ssems = (s0, s1, s2)

    lane = lax.iota(jnp.int32, 16)
    stages = []
    for k in range(3):
        pos = base + jnp.minimum(lane + 16 * k, limit)
        stages.append(
            pltpu.async_copy(idx_hbm.at[pos], idx_v.at[pl.ds(16 * k, 16)], gsems[k])
        )

    def gather(c, b):
        return pltpu.async_copy(
            table_hbm.at[idx_v.at[pl.ds(c * _CHUNK, _CHUNK)]],
            rows_v.at[b],
            gsems[b],
        )

    def scatter(c, b):
        return pltpu.async_copy(
            rows_v.at[b],
            out_hbm.at[pl.ds(t0 + c * _CHUNK, _CHUNK), row],
            ssems[b],
        )

    stages[0].wait()
    gathers = [gather(0, 0), gather(1, 1), None]
    stages[1].wait()
    gathers[2] = gather(2, 2)
    gathers[0].wait()
    scatters = [scatter(0, 0), None, None]
    stages[2].wait()
    gathers[1].wait()
    scatters[1] = scatter(1, 1)
    scatters[0].wait()
    gathers[0] = gather(3, 0)
    gathers[2].wait()
    scatters[2] = scatter(2, 2)
    scatters[1].wait()
    gathers[1] = gather(4, 1)
    gathers[0].wait()
    scatters[0] = scatter(3, 0)
    gathers[1].wait()

    @pl.when(part == 0)
    def _():
        s = pltpu.async_copy(
            rows_v.at[1],
            out_hbm.at[pl.ds(32, _CHUNK), row],
            ssems[1],
        )
        s.wait()

    @pl.when(part == 1)
    def _():
        s = pltpu.async_copy(
            rows_v.at[1, pl.ds(0, 5)],
            out_hbm.at[pl.ds(72, 5), row],
            ssems[1],
        )
        s.wait()

    scatters[0].wait()
    scatters[2].wait()


_mesh = plsc.VectorSubcoreMesh(core_axis_name="c", subcore_axis_name="s")

_emb_lookup = functools.partial(
    pl.kernel,
    mesh=_mesh,
    out_type=jax.ShapeDtypeStruct((_T, 16, D_MODEL), jnp.float32),
    scratch_types=[
        pltpu.VMEM((48,), jnp.int32),
        pltpu.VMEM((3, _CHUNK, D_MODEL), jnp.float32),
        pltpu.SemaphoreType.DMA,
        pltpu.SemaphoreType.DMA,
        pltpu.SemaphoreType.DMA,
        pltpu.SemaphoreType.DMA,
        pltpu.SemaphoreType.DMA,
        pltpu.SemaphoreType.DMA,
    ],
)(_emb_body)


@jax.jit
def kernel(test_input, emb_table):
    idx = test_input.reshape(-1).astype(jnp.int32)
    out_tb = _emb_lookup(idx, emb_table)
    return jnp.swapaxes(out_tb, 0, 1)
